# baseline (speedup 1.0000x reference)
import functools

import jax
import jax.numpy as jnp
from jax import lax
from jax.experimental import pallas as pl
from jax.experimental.pallas import tpu as pltpu
from jax.experimental.pallas import tpu_sc as plsc

_EPS = 1e-09
_N_TASKS = 1000
_N_TASKS_PAD = 1024
_N_SKILLS = 64
_N_SPLITS = 8
_BS = 16384
_D = _N_SKILLS * _N_SPLITS

_NC = 2
_NS = 16
_NW = _NC * _NS
_B_PER_W = _BS // _NW
_T_PER_S = _N_TASKS_PAD // _NS
_TCH = 32
_G = 32
_BCH = 128
_N_UNITS = _B_PER_W // _G
_UPC = _BCH // _G


def _normalize_rows(buf_v, n_rows):
    lanes = lax.iota(jnp.int32, 16)
    dnums = lax.GatherDimensionNumbers(
        offset_dims=(), collapsed_slice_dims=(0,), start_index_map=(0,)
    )

    def lane_perm(v, idx):
        return lax.gather(
            v,
            idx.reshape(16, 1),
            dnums,
            slice_sizes=(1,),
            mode=lax.GatherScatterMode.PROMISE_IN_BOUNDS,
        )

    def do_row(r, carry):
        for g in range(_N_SPLITS):
            base = g * _N_SKILLS
            vals = []
            for j in range(_N_SKILLS // 16):
                x = buf_v[r, pl.ds(base + j * 16, 16)]
                vals.append(1.0 / (1.0 + jnp.exp(-x)))
            tot = (vals[0] + vals[1]) + (vals[2] + vals[3])
            for k in (8, 4, 2, 1):
                tot = tot + lane_perm(tot, lanes ^ k)
            inv = 1.0 / (tot + _EPS)
            for j in range(_N_SKILLS // 16):
                buf_v[r, pl.ds(base + j * 16, 16)] = vals[j] * inv
        return carry

    lax.fori_loop(0, n_rows, do_row, 0)


def _transpose_unit(gbuf, tbuf, lane_base):
    biota = lax.iota(jnp.int32, 16)

    def body(k, carry):
        for s in range(_N_SPLITS):
            colv = biota * 0 + (s * _N_SKILLS + k)
            for bb in range(_G // 16):
                v = plsc.load_gather(gbuf, [biota + bb * 16, colv])
                tbuf[s, k, pl.ds(lane_base + bb * 16, 16)] = v
        return carry

    lax.fori_loop(0, _N_SKILLS, body, 0)


def _sc_body(idx_hbm, table_hbm, out_hbm, ptable_hbm, idx_v, gbuf_a, gbuf_b,
             tbuf, gsem_a, gsem_b, wsem):
    sid = lax.axis_index("s")
    cid = lax.axis_index("c")
    wid = sid * _NC + cid

    trow0 = sid * _T_PER_S
    for t in range(_T_PER_S // _TCH):
        rows = trow0 + t * _TCH
        pltpu.sync_copy(table_hbm.at[pl.ds(rows, _TCH)], gbuf_a)
        _normalize_rows(gbuf_a, _TCH)
        pltpu.sync_copy(gbuf_a, ptable_hbm.at[pl.ds(rows, _TCH)])
    plsc.subcore_barrier()

    base_row = wid * _B_PER_W
    pltpu.sync_copy(idx_hbm.at[pl.ds(base_row, _B_PER_W)], idx_v)

    gbufs = (gbuf_a, gbuf_b)
    gsems = (gsem_a, gsem_b)

    def issue_gather(u):
        return pltpu.async_copy(
            ptable_hbm.at[idx_v.at[pl.ds(u * _G, _G)]],
            gbufs[u % 2], gsems[u % 2],
        )

    wcp = None
    gcps = [issue_gather(0), None]
    for u in range(_N_UNITS):
        gcps[u % 2].wait()
        if u + 1 < _N_UNITS:
            gcps[(u + 1) % 2] = issue_gather(u + 1)
        if u % _UPC == 0 and wcp is not None:
            wcp.wait()
        _transpose_unit(gbufs[u % 2], tbuf, (u % _UPC) * _G)
        if u % _UPC == _UPC - 1:
            b0 = base_row + (u // _UPC) * _BCH
            wcp = pltpu.async_copy(
                tbuf, out_hbm.at[:, :, pl.ds(b0, _BCH)], wsem
            )
    wcp.wait()


@functools.partial(
    pl.kernel,
    mesh=plsc.VectorSubcoreMesh(core_axis_name="c", subcore_axis_name="s"),
    compiler_params=pltpu.CompilerParams(needs_layout_passes=False),
    out_type=(
        jax.ShapeDtypeStruct((_N_SPLITS, _N_SKILLS, _BS), jnp.float32),
        jax.ShapeDtypeStruct((_N_TASKS_PAD, _D), jnp.float32),
    ),
    scratch_types=[
        pltpu.VMEM((_B_PER_W,), jnp.int32),
        pltpu.VMEM((_G, _D), jnp.float32),
        pltpu.VMEM((_G, _D), jnp.float32),
        pltpu.VMEM((_N_SPLITS, _N_SKILLS, _BCH), jnp.float32),
        pltpu.SemaphoreType.DMA,
        pltpu.SemaphoreType.DMA,
        pltpu.SemaphoreType.DMA,
    ],
)
def _poly_selector(idx_hbm, table_hbm, out_hbm, ptable_hbm, idx_v, gbuf_a,
                   gbuf_b, tbuf, gsem_a, gsem_b, wsem):
    _sc_body(idx_hbm, table_hbm, out_hbm, ptable_hbm, idx_v, gbuf_a, gbuf_b,
             tbuf, gsem_a, gsem_b, wsem)


def kernel(routing_info, weights):
    idx = routing_info.reshape(-1).astype(jnp.int32)
    wpad = jnp.pad(weights, ((0, _N_TASKS_PAD - _N_TASKS), (0, 0)))
    out_t, _ = _poly_selector(idx, wpad)
    return jnp.transpose(out_t, (2, 0, 1))


# speedup vs baseline: 1.2725x; 1.2725x over previous
import functools

import jax
import jax.numpy as jnp
from jax import lax
from jax.experimental import pallas as pl
from jax.experimental.pallas import tpu as pltpu
from jax.experimental.pallas import tpu_sc as plsc

_EPS = 1e-09
_N_TASKS = 1000
_N_TASKS_PAD = 1024
_N_SKILLS = 64
_N_SPLITS = 8
_BS = 16384
_D = _N_SKILLS * _N_SPLITS

_NC = 2
_NS = 16
_NW = _NC * _NS
_B_PER_W = _BS // _NW
_T_PER_S = _N_TASKS_PAD // _NS
_TCH = 32
_G = 32
_BCH = 128
_N_UNITS = _B_PER_W // _G
_UPC = _BCH // _G


def _normalize_rows(buf_v, n_rows):
    lanes = lax.iota(jnp.int32, 16)
    dnums = lax.GatherDimensionNumbers(
        offset_dims=(), collapsed_slice_dims=(0,), start_index_map=(0,)
    )

    def lane_perm(v, idx):
        return lax.gather(
            v,
            idx.reshape(16, 1),
            dnums,
            slice_sizes=(1,),
            mode=lax.GatherScatterMode.PROMISE_IN_BOUNDS,
        )

    def do_row(r, carry):
        for g in range(_N_SPLITS):
            base = g * _N_SKILLS
            vals = []
            for j in range(_N_SKILLS // 16):
                x = buf_v[r, pl.ds(base + j * 16, 16)]
                vals.append(1.0 / (1.0 + jnp.exp(-x)))
            tot = (vals[0] + vals[1]) + (vals[2] + vals[3])
            for k in (8, 4, 2, 1):
                tot = tot + lane_perm(tot, lanes ^ k)
            inv = 1.0 / (tot + _EPS)
            for j in range(_N_SKILLS // 16):
                buf_v[r, pl.ds(base + j * 16, 16)] = vals[j] * inv
        return carry

    lax.fori_loop(0, n_rows, do_row, 0)


def _transpose_unit(gbuf, tbuf, lane_base):
    biota = lax.iota(jnp.int32, 16)
    rows = (biota, biota + 16)
    zero = biota * 0

    @plsc.parallel_loop(0, _N_SKILLS, unroll=4)
    def _k_loop(k):
        colvs = [zero + (s * _N_SKILLS + k) for s in range(_N_SPLITS)]
        vals = [
            plsc.load_gather(gbuf, [rows[bb], colvs[s]])
            for s in range(_N_SPLITS)
            for bb in range(_G // 16)
        ]
        i = 0
        for s in range(_N_SPLITS):
            for bb in range(_G // 16):
                tbuf[s, k, pl.ds(lane_base + bb * 16, 16)] = vals[i]
                i += 1


def _sc_body(idx_hbm, table_hbm, out_hbm, ptable_hbm, idx_v, gbuf_a, gbuf_b,
             tbuf, gsem_a, gsem_b, wsem):
    sid = lax.axis_index("s")
    cid = lax.axis_index("c")
    wid = sid * _NC + cid

    trow0 = sid * _T_PER_S
    for t in range(_T_PER_S // _TCH):
        rows = trow0 + t * _TCH
        pltpu.sync_copy(table_hbm.at[pl.ds(rows, _TCH)], gbuf_a)
        _normalize_rows(gbuf_a, _TCH)
        pltpu.sync_copy(gbuf_a, ptable_hbm.at[pl.ds(rows, _TCH)])
    plsc.subcore_barrier()

    base_row = wid * _B_PER_W
    pltpu.sync_copy(idx_hbm.at[pl.ds(base_row, _B_PER_W)], idx_v)

    gbufs = (gbuf_a, gbuf_b)
    gsems = (gsem_a, gsem_b)

    def issue_gather(u):
        return pltpu.async_copy(
            ptable_hbm.at[idx_v.at[pl.ds(u * _G, _G)]],
            gbufs[u % 2], gsems[u % 2],
        )

    wcp = None
    gcps = [issue_gather(0), None]
    for u in range(_N_UNITS):
        gcps[u % 2].wait()
        if u + 1 < _N_UNITS:
            gcps[(u + 1) % 2] = issue_gather(u + 1)
        if u % _UPC == 0 and wcp is not None:
            wcp.wait()
        _transpose_unit(gbufs[u % 2], tbuf, (u % _UPC) * _G)
        if u % _UPC == _UPC - 1:
            b0 = base_row + (u // _UPC) * _BCH
            wcp = pltpu.async_copy(
                tbuf, out_hbm.at[:, :, pl.ds(b0, _BCH)], wsem
            )
    wcp.wait()


@functools.partial(
    pl.kernel,
    mesh=plsc.VectorSubcoreMesh(core_axis_name="c", subcore_axis_name="s"),
    compiler_params=pltpu.CompilerParams(needs_layout_passes=False),
    out_type=(
        jax.ShapeDtypeStruct((_N_SPLITS, _N_SKILLS, _BS), jnp.float32),
        jax.ShapeDtypeStruct((_N_TASKS_PAD, _D), jnp.float32),
    ),
    scratch_types=[
        pltpu.VMEM((_B_PER_W,), jnp.int32),
        pltpu.VMEM((_G, _D), jnp.float32),
        pltpu.VMEM((_G, _D), jnp.float32),
        pltpu.VMEM((_N_SPLITS, _N_SKILLS, _BCH), jnp.float32),
        pltpu.SemaphoreType.DMA,
        pltpu.SemaphoreType.DMA,
        pltpu.SemaphoreType.DMA,
    ],
)
def _poly_selector(idx_hbm, table_hbm, out_hbm, ptable_hbm, idx_v, gbuf_a,
                   gbuf_b, tbuf, gsem_a, gsem_b, wsem):
    _sc_body(idx_hbm, table_hbm, out_hbm, ptable_hbm, idx_v, gbuf_a, gbuf_b,
             tbuf, gsem_a, gsem_b, wsem)


def kernel(routing_info, weights):
    idx = routing_info.reshape(-1).astype(jnp.int32)
    wpad = jnp.pad(weights, ((0, _N_TASKS_PAD - _N_TASKS), (0, 0)))
    out_t, _ = _poly_selector(idx, wpad)
    return jnp.transpose(out_t, (2, 0, 1))


# speedup vs baseline: 3.6881x; 2.8984x over previous
import functools

import jax
import jax.numpy as jnp
from jax import lax
from jax.experimental import pallas as pl
from jax.experimental.pallas import tpu as pltpu
from jax.experimental.pallas import tpu_sc as plsc

_EPS = 1e-09
_N_TASKS = 1000
_N_TASKS_PAD = 1024
_N_SKILLS = 64
_N_SPLITS = 8
_BS = 16384
_D = _N_SKILLS * _N_SPLITS

_NC = 2
_NS = 16
_NW = _NC * _NS
_B_PER_W = _BS // _NW
_T_PER_S = _N_TASKS_PAD // _NS
_TCH = 32
_G = 32
_BCH = 128
_N_PAIRS = _B_PER_W // (2 * _G)


def _normalize_rows(buf_v, n_rows):
    lanes = lax.iota(jnp.int32, 16)
    dnums = lax.GatherDimensionNumbers(
        offset_dims=(), collapsed_slice_dims=(0,), start_index_map=(0,)
    )

    def lane_perm(v, idx):
        return lax.gather(
            v,
            idx.reshape(16, 1),
            dnums,
            slice_sizes=(1,),
            mode=lax.GatherScatterMode.PROMISE_IN_BOUNDS,
        )

    def do_row(r, carry):
        for g in range(_N_SPLITS):
            base = g * _N_SKILLS
            vals = []
            for j in range(_N_SKILLS // 16):
                x = buf_v[r, pl.ds(base + j * 16, 16)]
                vals.append(1.0 / (1.0 + jnp.exp(-x)))
            tot = (vals[0] + vals[1]) + (vals[2] + vals[3])
            for k in (8, 4, 2, 1):
                tot = tot + lane_perm(tot, lanes ^ k)
            inv = 1.0 / (tot + _EPS)
            for j in range(_N_SKILLS // 16):
                buf_v[r, pl.ds(base + j * 16, 16)] = vals[j] * inv
        return carry

    lax.fori_loop(0, n_rows, do_row, 0)


def _transpose_unit(gbuf, tbuf, lane_base, biota):
    for s in range(_N_SPLITS):
        sv = biota * 0 + s

        @plsc.parallel_loop(0, _N_SKILLS, unroll=4)
        def _diag_loop(i):
            k0 = (i >> 4) * 16
            rot = (biota + i) & 15
            kv = rot + k0
            for bb in range(_G // 16):
                rowv = biota + bb * 16
                bv = biota + (lane_base + bb * 16)
                v = plsc.load_gather(gbuf, [rowv, kv + s * _N_SKILLS])
                plsc.store_scatter(tbuf, [sv, kv, bv], v)


def _sc_body(idx_hbm, table_hbm, out_hbm, ptable_hbm, idx_v, gbuf_a, gbuf_b,
             tbuf, gsem_a, gsem_b, wsem):
    sid = lax.axis_index("s")
    cid = lax.axis_index("c")
    wid = sid * _NC + cid
    biota = lax.iota(jnp.int32, 16)

    trow0 = sid * _T_PER_S
    for t in range(_T_PER_S // _TCH):
        rows = trow0 + t * _TCH
        pltpu.sync_copy(table_hbm.at[pl.ds(rows, _TCH)], gbuf_a)
        _normalize_rows(gbuf_a, _TCH)
        pltpu.sync_copy(gbuf_a, ptable_hbm.at[pl.ds(rows, _TCH)])
    plsc.subcore_barrier()

    base_row = wid * _B_PER_W
    pltpu.sync_copy(idx_hbm.at[pl.ds(base_row, _B_PER_W)], idx_v)

    def issue_gather(u, gbuf, gsem):
        pltpu.async_copy(
            ptable_hbm.at[idx_v.at[pl.ds(u * _G, _G)]], gbuf, gsem
        )

    def wait_gather(gbuf, gsem):
        pltpu.make_async_copy(
            ptable_hbm.at[pl.ds(0, _G)], gbuf, gsem
        ).wait()

    def drain_write():
        pltpu.make_async_copy(
            out_hbm.at[:, :, pl.ds(0, _BCH)], tbuf, wsem
        ).wait()

    issue_gather(0, gbuf_a, gsem_a)

    def do_pair(p, carry):
        u0 = 2 * p
        lane0 = (p % 2) * 64
        wait_gather(gbuf_a, gsem_a)
        issue_gather(u0 + 1, gbuf_b, gsem_b)
        @pl.when(jnp.logical_and(p % 2 == 0, p > 0))
        def _():
            drain_write()

        _transpose_unit(gbuf_a, tbuf, lane0, biota)
        wait_gather(gbuf_b, gsem_b)

        @pl.when(p < _N_PAIRS - 1)
        def _():
            issue_gather(u0 + 2, gbuf_a, gsem_a)

        _transpose_unit(gbuf_b, tbuf, lane0 + _G, biota)

        @pl.when(p % 2 == 1)
        def _():
            b0 = base_row + (p // 2) * _BCH
            pltpu.async_copy(tbuf, out_hbm.at[:, :, pl.ds(b0, _BCH)], wsem)

        return carry

    lax.fori_loop(0, _N_PAIRS, do_pair, 0)
    drain_write()


@functools.partial(
    pl.kernel,
    mesh=plsc.VectorSubcoreMesh(core_axis_name="c", subcore_axis_name="s"),
    compiler_params=pltpu.CompilerParams(needs_layout_passes=False),
    out_type=(
        jax.ShapeDtypeStruct((_N_SPLITS, _N_SKILLS, _BS), jnp.float32),
        jax.ShapeDtypeStruct((_N_TASKS_PAD, _D), jnp.float32),
    ),
    scratch_types=[
        pltpu.VMEM((_B_PER_W,), jnp.int32),
        pltpu.VMEM((_G, _D), jnp.float32),
        pltpu.VMEM((_G, _D), jnp.float32),
        pltpu.VMEM((_N_SPLITS, _N_SKILLS, _BCH), jnp.float32),
        pltpu.SemaphoreType.DMA,
        pltpu.SemaphoreType.DMA,
        pltpu.SemaphoreType.DMA,
    ],
)
def _poly_selector(idx_hbm, table_hbm, out_hbm, ptable_hbm, idx_v, gbuf_a,
                   gbuf_b, tbuf, gsem_a, gsem_b, wsem):
    _sc_body(idx_hbm, table_hbm, out_hbm, ptable_hbm, idx_v, gbuf_a, gbuf_b,
             tbuf, gsem_a, gsem_b, wsem)


def kernel(routing_info, weights):
    idx = routing_info.reshape(-1).astype(jnp.int32)
    wpad = jnp.pad(weights, ((0, _N_TASKS_PAD - _N_TASKS), (0, 0)))
    out_t, _ = _poly_selector(idx, wpad)
    return jnp.transpose(out_t, (2, 0, 1))


# speedup vs baseline: 3.6909x; 1.0008x over previous
import functools

import jax
import jax.numpy as jnp
from jax import lax
from jax.experimental import pallas as pl
from jax.experimental.pallas import tpu as pltpu
from jax.experimental.pallas import tpu_sc as plsc

_EPS = 1e-09
_N_TASKS = 1000
_N_TASKS_PAD = 1024
_N_SKILLS = 64
_N_SPLITS = 8
_BS = 16384
_D = _N_SKILLS * _N_SPLITS

_NC = 2
_NS = 16
_NW = _NC * _NS
_B_PER_W = _BS // _NW
_T_PER_S = _N_TASKS_PAD // _NS
_TCH = 32
_G = 32
_BCH = 128
_N_PAIRS = _B_PER_W // (2 * _G)


def _normalize_rows(buf_v, n_rows):
    lanes = lax.iota(jnp.int32, 16)
    dnums = lax.GatherDimensionNumbers(
        offset_dims=(), collapsed_slice_dims=(0,), start_index_map=(0,)
    )

    def lane_perm(v, idx):
        return lax.gather(
            v,
            idx.reshape(16, 1),
            dnums,
            slice_sizes=(1,),
            mode=lax.GatherScatterMode.PROMISE_IN_BOUNDS,
        )

    def do_row(r, carry):
        for g in range(_N_SPLITS):
            base = g * _N_SKILLS
            vals = []
            for j in range(_N_SKILLS // 16):
                x = buf_v[r, pl.ds(base + j * 16, 16)]
                vals.append(1.0 / (1.0 + jnp.exp(-x)))
            tot = (vals[0] + vals[1]) + (vals[2] + vals[3])
            for k in (8, 4, 2, 1):
                tot = tot + lane_perm(tot, lanes ^ k)
            inv = 1.0 / (tot + _EPS)
            for j in range(_N_SKILLS // 16):
                buf_v[r, pl.ds(base + j * 16, 16)] = vals[j] * inv
        return carry

    lax.fori_loop(0, n_rows, do_row, 0)


def _transpose_unit(gbuf, tbuf, lane_base, biota):
    for s in range(_N_SPLITS):
        sv = biota * 0 + s

        @plsc.parallel_loop(0, _N_SKILLS, unroll=4)
        def _diag_loop(i):
            k0 = (i >> 4) * 16
            rot = (biota + i) & 15
            kv = rot + k0
            for bb in range(_G // 16):
                rowv = biota + bb * 16
                bv = biota + (lane_base + bb * 16)
                v = plsc.load_gather(gbuf, [rowv, kv + s * _N_SKILLS])
                plsc.store_scatter(tbuf, [sv, kv, bv], v)


def _sc_body(idx_hbm, table_hbm, out_hbm, ptable_hbm, idx_v, gbuf_a, gbuf_b,
             tbuf, gsem_a, gsem_b, wsem):
    sid = lax.axis_index("s")
    cid = lax.axis_index("c")
    wid = sid * _NC + cid
    biota = lax.iota(jnp.int32, 16)

    trow0 = sid * _T_PER_S
    for t in range(_T_PER_S // _TCH):
        rows = trow0 + t * _TCH
        pltpu.sync_copy(table_hbm.at[pl.ds(rows, _TCH)], gbuf_a)
        _normalize_rows(gbuf_a, _TCH)
        pltpu.sync_copy(gbuf_a, ptable_hbm.at[pl.ds(rows, _TCH)])
    plsc.subcore_barrier()

    base_row = wid * _B_PER_W
    pltpu.sync_copy(idx_hbm.at[pl.ds(base_row, _B_PER_W)], idx_v)

    def issue_gather(u, gbuf, gsem):
        pltpu.async_copy(
            ptable_hbm.at[idx_v.at[pl.ds(u * _G, _G)]], gbuf, gsem
        )

    def wait_gather(gbuf, gsem):
        pltpu.make_async_copy(
            ptable_hbm.at[pl.ds(0, _G)], gbuf, gsem
        ).wait()

    def drain_write():
        pltpu.make_async_copy(
            out_hbm.at[:, :, pl.ds(0, _BCH)], tbuf, wsem
        ).wait()

    issue_gather(0, gbuf_a, gsem_a)

    def do_pair(p, carry):
        u0 = 2 * p
        lane0 = (p % 2) * 64
        wait_gather(gbuf_a, gsem_a)
        issue_gather(u0 + 1, gbuf_b, gsem_b)
        _transpose_unit(gbuf_a, tbuf, lane0, biota)
        wait_gather(gbuf_b, gsem_b)

        @pl.when(p < _N_PAIRS - 1)
        def _():
            issue_gather(u0 + 2, gbuf_a, gsem_a)

        _transpose_unit(gbuf_b, tbuf, lane0 + _G, biota)

        @pl.when(p % 2 == 1)
        def _():
            b0 = base_row + (p // 2) * _BCH
            pltpu.async_copy(tbuf, out_hbm.at[:, :, pl.ds(b0, _BCH)], wsem)
            drain_write()

        return carry

    lax.fori_loop(0, _N_PAIRS, do_pair, 0)


@functools.partial(
    pl.kernel,
    mesh=plsc.VectorSubcoreMesh(core_axis_name="c", subcore_axis_name="s"),
    compiler_params=pltpu.CompilerParams(needs_layout_passes=False),
    out_type=(
        jax.ShapeDtypeStruct((_N_SPLITS, _N_SKILLS, _BS), jnp.float32),
        jax.ShapeDtypeStruct((_N_TASKS_PAD, _D), jnp.float32),
    ),
    scratch_types=[
        pltpu.VMEM((_B_PER_W,), jnp.int32),
        pltpu.VMEM((_G, _D), jnp.float32),
        pltpu.VMEM((_G, _D), jnp.float32),
        pltpu.VMEM((_N_SPLITS, _N_SKILLS, _BCH), jnp.float32),
        pltpu.SemaphoreType.DMA,
        pltpu.SemaphoreType.DMA,
        pltpu.SemaphoreType.DMA,
    ],
)
def _poly_selector(idx_hbm, table_hbm, out_hbm, ptable_hbm, idx_v, gbuf_a,
                   gbuf_b, tbuf, gsem_a, gsem_b, wsem):
    _sc_body(idx_hbm, table_hbm, out_hbm, ptable_hbm, idx_v, gbuf_a, gbuf_b,
             tbuf, gsem_a, gsem_b, wsem)


def kernel(routing_info, weights):
    idx = routing_info.reshape(-1).astype(jnp.int32)
    wpad = jnp.pad(weights, ((0, _N_TASKS_PAD - _N_TASKS), (0, 0)))
    out_t, _ = _poly_selector(idx, wpad)
    return jnp.transpose(out_t, (2, 0, 1))


# speedup vs baseline: 3.8307x; 1.0379x over previous
import functools

import jax
import jax.numpy as jnp
from jax import lax
from jax.experimental import pallas as pl
from jax.experimental.pallas import tpu as pltpu
from jax.experimental.pallas import tpu_sc as plsc

_EPS = 1e-09
_N_TASKS = 1000
_N_TASKS_PAD = 1024
_N_SKILLS = 64
_N_SPLITS = 8
_BS = 16384
_D = _N_SKILLS * _N_SPLITS

_NC = 2
_NS = 16
_NW = _NC * _NS
_B_PER_W = _BS // _NW
_T_PER_S = _N_TASKS_PAD // _NS
_TCH = 32
_G = 32
_BCH = 128
_N_PAIRS = _B_PER_W // (2 * _G)


def _normalize_rows(buf_v, n_rows):
    lanes = lax.iota(jnp.int32, 16)
    dnums = lax.GatherDimensionNumbers(
        offset_dims=(), collapsed_slice_dims=(0,), start_index_map=(0,)
    )

    def lane_perm(v, idx):
        return lax.gather(
            v,
            idx.reshape(16, 1),
            dnums,
            slice_sizes=(1,),
            mode=lax.GatherScatterMode.PROMISE_IN_BOUNDS,
        )

    def do_row(r, carry):
        for g in range(_N_SPLITS):
            base = g * _N_SKILLS
            vals = []
            for j in range(_N_SKILLS // 16):
                x = buf_v[r, pl.ds(base + j * 16, 16)]
                vals.append(1.0 / (1.0 + jnp.exp(-x)))
            tot = (vals[0] + vals[1]) + (vals[2] + vals[3])
            for k in (8, 4, 2, 1):
                tot = tot + lane_perm(tot, lanes ^ k)
            inv = 1.0 / (tot + _EPS)
            for j in range(_N_SKILLS // 16):
                buf_v[r, pl.ds(base + j * 16, 16)] = vals[j] * inv
        return carry

    lax.fori_loop(0, n_rows, do_row, 0)


def _transpose_unit(gbuf, tbuf, lane_base, biota):
    for s in range(_N_SPLITS):
        sv = biota * 0 + s

        @plsc.parallel_loop(0, _N_SKILLS, unroll=4)
        def _diag_loop(i):
            k0 = (i >> 4) * 16
            rot = (biota + i) & 15
            kv = rot + k0
            for bb in range(_G // 16):
                rowv = biota + bb * 16
                bv = biota + (lane_base + bb * 16)
                v = plsc.load_gather(gbuf, [rowv, kv + s * _N_SKILLS])
                plsc.store_scatter(tbuf, [sv, kv, bv], v)


def _sc_body(idx_hbm, table_hbm, out_hbm, ptable_hbm, idx_v, gbuf_a, gbuf_b,
             tbuf, gsem_a, gsem_b, wsem):
    sid = lax.axis_index("s")
    cid = lax.axis_index("c")
    wid = sid * _NC + cid
    biota = lax.iota(jnp.int32, 16)

    trow0 = sid * _T_PER_S
    rows0 = jnp.minimum(trow0, _N_TASKS - _TCH)
    rows1 = jnp.minimum(trow0 + _TCH, _N_TASKS - _TCH)
    in0 = pltpu.async_copy(table_hbm.at[pl.ds(rows0, _TCH)], gbuf_a, gsem_a)
    in1 = pltpu.async_copy(table_hbm.at[pl.ds(rows1, _TCH)], gbuf_b, gsem_b)
    in0.wait()
    _normalize_rows(gbuf_a, _TCH)
    out0 = pltpu.async_copy(gbuf_a, ptable_hbm.at[pl.ds(rows0, _TCH)], gsem_a)
    in1.wait()
    _normalize_rows(gbuf_b, _TCH)
    out0.wait()
    pltpu.sync_copy(gbuf_b, ptable_hbm.at[pl.ds(rows1, _TCH)])
    plsc.subcore_barrier()

    base_row = wid * _B_PER_W
    pltpu.sync_copy(idx_hbm.at[pl.ds(base_row, _B_PER_W)], idx_v)

    def issue_gather(u, gbuf, gsem):
        pltpu.async_copy(
            ptable_hbm.at[idx_v.at[pl.ds(u * _G, _G)]], gbuf, gsem
        )

    def wait_gather(gbuf, gsem):
        pltpu.make_async_copy(
            ptable_hbm.at[pl.ds(0, _G)], gbuf, gsem
        ).wait()

    def drain_write():
        pltpu.make_async_copy(
            out_hbm.at[:, :, pl.ds(0, _BCH)], tbuf, wsem
        ).wait()

    issue_gather(0, gbuf_a, gsem_a)

    def do_pair(p, carry):
        u0 = 2 * p
        lane0 = (p % 2) * 64
        wait_gather(gbuf_a, gsem_a)
        issue_gather(u0 + 1, gbuf_b, gsem_b)
        _transpose_unit(gbuf_a, tbuf, lane0, biota)
        wait_gather(gbuf_b, gsem_b)

        @pl.when(p < _N_PAIRS - 1)
        def _():
            issue_gather(u0 + 2, gbuf_a, gsem_a)

        _transpose_unit(gbuf_b, tbuf, lane0 + _G, biota)

        @pl.when(p % 2 == 1)
        def _():
            b0 = base_row + (p // 2) * _BCH
            pltpu.async_copy(tbuf, out_hbm.at[:, :, pl.ds(b0, _BCH)], wsem)
            drain_write()

        return carry

    lax.fori_loop(0, _N_PAIRS, do_pair, 0)


@functools.partial(
    pl.kernel,
    mesh=plsc.VectorSubcoreMesh(core_axis_name="c", subcore_axis_name="s"),
    compiler_params=pltpu.CompilerParams(needs_layout_passes=False),
    out_type=(
        jax.ShapeDtypeStruct((_N_SPLITS, _N_SKILLS, _BS), jnp.float32),
        jax.ShapeDtypeStruct((_N_TASKS, _D), jnp.float32),
    ),
    scratch_types=[
        pltpu.VMEM((_B_PER_W,), jnp.int32),
        pltpu.VMEM((_G, _D), jnp.float32),
        pltpu.VMEM((_G, _D), jnp.float32),
        pltpu.VMEM((_N_SPLITS, _N_SKILLS, _BCH), jnp.float32),
        pltpu.SemaphoreType.DMA,
        pltpu.SemaphoreType.DMA,
        pltpu.SemaphoreType.DMA,
    ],
)
def _poly_selector(idx_hbm, table_hbm, out_hbm, ptable_hbm, idx_v, gbuf_a,
                   gbuf_b, tbuf, gsem_a, gsem_b, wsem):
    _sc_body(idx_hbm, table_hbm, out_hbm, ptable_hbm, idx_v, gbuf_a, gbuf_b,
             tbuf, gsem_a, gsem_b, wsem)


def kernel(routing_info, weights):
    idx = routing_info.reshape(-1).astype(jnp.int32)
    out_t, _ = _poly_selector(idx, weights)
    return jnp.transpose(out_t, (2, 0, 1))
